# baseline (speedup 1.0000x reference)
import functools

import jax
import jax.numpy as jnp
from jax import lax
from jax.experimental import pallas as pl
from jax.experimental.pallas import tpu as pltpu
from jax.experimental.pallas import tpu_sc as plsc

_DIM = 32
_LANES = 16
_CH = 1024
_SROW = 128
_SEG = 2048
_SBITS = 15


def _make_kernels(batch, size):
    info = plsc.get_sparse_core_info()
    nc, ns = info.num_cores, info.num_subcores
    nw = nc * ns
    b_per_w = batch // nw
    tail = size % _SROW
    main = size - tail
    rng = 244 * _SROW
    extra = main - nw * rng
    n_full = rng // _CH
    half = rng - n_full * _CH
    stage_n = batch // 2
    dummy = batch
    smask = (1 << _SBITS) - 1

    mesh = plsc.VectorSubcoreMesh(core_axis_name="c", subcore_axis_name="s")

    @functools.partial(
        pl.kernel,
        mesh=mesh,
        out_type=(jax.ShapeDtypeStruct((batch + 1, _SROW), jnp.float32),
                  jax.ShapeDtypeStruct((batch + 1, _SROW), jnp.float32)),
        compiler_params=pltpu.CompilerParams(needs_layout_passes=False),
        scratch_types=[
            pltpu.VMEM((stage_n,), jnp.int32),
            pltpu.VMEM((batch,), jnp.int32),
            pltpu.VMEM((batch,), jnp.int32),
            pltpu.VMEM((2, _DIM, _CH), jnp.float32),
            pltpu.VMEM((_LANES, _SROW), jnp.float32),
            pltpu.VMEM((_LANES, _SROW), jnp.float32),
            pltpu.VMEM((_LANES,), jnp.int32),
            pltpu.VMEM((_LANES,), jnp.int32),
            pltpu.VMEM((_SEG,), jnp.int32),
            pltpu.SemaphoreType.DMA,
            pltpu.SemaphoreType.DMA,
            pltpu.SemaphoreType.DMA,
        ],
    )
    def scan_kernel(left_hbm, right_hbm, tablet_hbm, aux_hbm,
                    gl_hbm, gr_hbm,
                    stage_v, cl_v, cr_v, cb_v, rsA_v, rsB_v,
                    slA_v, slB_v, mini_v, sem_in, semA, semB):
        w = lax.axis_index("s") * nc + lax.axis_index("c")
        last = w == nw - 1
        lo = w * rng
        hi = jnp.where(last, jnp.int32(size), lo + rng)
        lanev = lax.iota(jnp.int32, _LANES)

        def chunk_src(j):
            off = pl.multiple_of(lo + j * _CH, _SROW)
            return tablet_hbm.at[:, pl.ds(off, _CH)]

        pltpu.async_copy(chunk_src(0), cb_v.at[0], sem_in)
        pltpu.async_copy(chunk_src(1), cb_v.at[1], sem_in)

        def compact(src_hbm, list_v):
            def stage_body(t, n):
                pltpu.sync_copy(src_hbm.at[pl.ds(t * stage_n, stage_n)],
                                stage_v)

                def body(v, n):
                    at = v * (4 * _LANES)
                    for k in range(4):
                        vec = stage_v[pl.ds(at + k * _LANES, _LANES)]
                        m = (vec >= lo) & (vec < hi)
                        slot = t * stage_n + at + k * _LANES + lanev
                        packed = ((vec - lo) << _SBITS) | slot
                        pos = n + plsc.cumsum(m.astype(jnp.int32)) - 1
                        plsc.store_scatter(list_v, [pos], packed, mask=m)
                        n = n + plsc.all_reduce_population_count(m)
                    return n

                return lax.fori_loop(0, stage_n // (4 * _LANES), body, n)

            n = lax.fori_loop(0, batch // stage_n, stage_body,
                              jnp.zeros((_LANES,), jnp.int32))
            return n[0]

        n_l = compact(left_hbm, cl_v)
        n_r = compact(right_hbm, cr_v)

        def rescan(list_v, n_list, g_hbm, buf, base_rel, width, carry):
            def seg_body(s, carry):
                seg = s * _SEG

                def match_body(v, q):
                    at = seg + v * (4 * _LANES)
                    for k in range(4):
                        vec = list_v[pl.ds(at + k * _LANES, _LANES)]
                        rel = vec >> _SBITS
                        valid = (at + k * _LANES + lanev) < n_list
                        m = ((rel >= base_rel) & (rel < base_rel + width)
                             & valid)
                        pos = q + plsc.cumsum(m.astype(jnp.int32)) - 1
                        plsc.store_scatter(mini_v, [pos], vec, mask=m)
                        q = q + plsc.all_reduce_population_count(m)
                    return q

                left_n = n_list - seg
                n_iters = jnp.clip(
                    (left_n + 4 * _LANES - 1) // (4 * _LANES),
                    0, _SEG // (4 * _LANES))
                q = lax.fori_loop(0, n_iters, match_body,
                                  jnp.zeros((_LANES,), jnp.int32))[0]

                def extract_body(gq, carry):
                    sA, sB = carry
                    packed = mini_v[pl.ds(gq * _LANES, _LANES)]
                    valid = (gq * _LANES + lanev) < q
                    cols = jnp.where(valid, (packed >> _SBITS) - base_rel,
                                     0)
                    slot = jnp.where(valid, packed & smask, dummy)
                    even = ((sA + sB) % 2) == 0

                    def extract(rs_v, sl_v, sem):
                        for c in range(_DIM):
                            vals = plsc.load_gather(
                                buf,
                                [jnp.full((_LANES,), c, jnp.int32), cols])
                            plsc.store_scatter(
                                rs_v,
                                [lanev, jnp.full((_LANES,), c, jnp.int32)],
                                vals)
                        sl_v[...] = slot
                        pltpu.async_copy(rs_v, g_hbm.at[sl_v], sem)

                    @pl.when(even)
                    def _():
                        @pl.when(sA >= 1)
                        def _():
                            pltpu.make_async_copy(
                                rsA_v, g_hbm.at[pl.ds(0, _LANES)],
                                semA).wait()
                        extract(rsA_v, slA_v, semA)

                    @pl.when(jnp.logical_not(even))
                    def _():
                        @pl.when(sB >= 1)
                        def _():
                            pltpu.make_async_copy(
                                rsB_v, g_hbm.at[pl.ds(0, _LANES)],
                                semB).wait()
                        extract(rsB_v, slB_v, semB)

                    sA = sA + jnp.where(even, 1, 0)
                    sB = sB + jnp.where(even, 0, 1)
                    return (sA, sB)

                n_groups = (q + _LANES - 1) // _LANES
                return lax.fori_loop(0, n_groups, extract_body, carry)

            n_segs = (n_list + _SEG - 1) // _SEG
            return lax.fori_loop(0, n_segs, seg_body, carry)

        def chunk_body(j, carry):
            base_rel = j * _CH
            pltpu.make_async_copy(chunk_src(0), cb_v.at[j % 2],
                                  sem_in).wait()
            carry = rescan(cl_v, n_l, gl_hbm, cb_v.at[j % 2],
                           base_rel, _CH, carry)
            carry = rescan(cr_v, n_r, gr_hbm, cb_v.at[j % 2],
                           base_rel, _CH, carry)

            @pl.when(j + 2 < n_full)
            def _():
                pltpu.async_copy(chunk_src(j + 2), cb_v.at[j % 2], sem_in)

            return carry

        carry = lax.fori_loop(0, n_full, chunk_body,
                              (jnp.int32(0), jnp.int32(0)))

        off = pl.multiple_of(lo + n_full * _CH, _SROW)
        pltpu.sync_copy(tablet_hbm.at[:, pl.ds(off, half)],
                        cb_v.at[0].at[:, pl.ds(0, half)])
        carry = rescan(cl_v, n_l, gl_hbm, cb_v.at[0],
                       jnp.int32(n_full * _CH), jnp.int32(half), carry)
        carry = rescan(cr_v, n_r, gr_hbm, cb_v.at[0],
                       jnp.int32(n_full * _CH), jnp.int32(half), carry)

        @pl.when(last)
        def _():
            pltpu.sync_copy(
                tablet_hbm.at[:, pl.ds(pl.multiple_of(nw * rng, _SROW),
                                       extra)],
                cb_v.at[1].at[:, pl.ds(0, extra)])
            pltpu.sync_copy(aux_hbm, cb_v.at[0].at[:, pl.ds(0, _SROW)])

        ex_carry = carry
        for g_hbm, list_v, n_list in ((gl_hbm, cl_v, n_l),
                                      (gr_hbm, cr_v, n_r)):
            ex_carry = rescan(list_v, jnp.where(last, n_list, 0), g_hbm,
                              cb_v.at[1], jnp.int32(rng), jnp.int32(extra),
                              ex_carry)
            ex_carry = rescan(list_v, jnp.where(last, n_list, 0), g_hbm,
                              cb_v.at[0], jnp.int32(rng + extra),
                              jnp.int32(tail), ex_carry)
        sA, sB = ex_carry

        @pl.when(sA >= 1)
        def _():
            pltpu.make_async_copy(rsA_v, gl_hbm.at[pl.ds(0, _LANES)],
                                  semA).wait()

        @pl.when(sB >= 1)
        def _():
            pltpu.make_async_copy(rsB_v, gl_hbm.at[pl.ds(0, _LANES)],
                                  semB).wait()

    n_blocks = b_per_w // 128

    @functools.partial(
        pl.kernel,
        mesh=mesh,
        out_type=jax.ShapeDtypeStruct((nw, _LANES), jnp.float32),
        compiler_params=pltpu.CompilerParams(needs_layout_passes=False),
        scratch_types=[
            pltpu.VMEM((b_per_w,), jnp.float32),
            pltpu.VMEM((128, _SROW), jnp.float32),
            pltpu.VMEM((128, _SROW), jnp.float32),
            pltpu.VMEM((128, _SROW), jnp.float32),
            pltpu.VMEM((128, _SROW), jnp.float32),
            pltpu.VMEM((_LANES,), jnp.float32),
            pltpu.SemaphoreType.DMA,
        ],
    )
    def dot_kernel(gl_hbm, gr_hbm, cov_hbm, out_hbm,
                   cov_v, la_v, lb_v, ra_v, rb_v, loss_v, sem):
        w = lax.axis_index("s") * nc + lax.axis_index("c")
        pltpu.sync_copy(cov_hbm.at[w], cov_v)
        lbufs = (la_v, lb_v)
        rbufs = (ra_v, rb_v)
        lanev = lax.iota(jnp.int32, _LANES)

        def fire(j):
            base = w * b_per_w + j * 128
            return (pltpu.async_copy(gl_hbm.at[pl.ds(base, 128)],
                                     lbufs[j % 2], sem),
                    pltpu.async_copy(gr_hbm.at[pl.ds(base, 128)],
                                     rbufs[j % 2], sem))

        def make_group_body(lbuf, rbuf, j):
            def group_body(g, loss):
                row = g * _LANES + lanev
                acc = jnp.zeros((_LANES,), jnp.float32)
                for c in range(_DIM):
                    col = jnp.full((_LANES,), c, jnp.int32)
                    lv = plsc.load_gather(lbuf, [row, col])
                    rv = plsc.load_gather(rbuf, [row, col])
                    acc = acc + lv * rv
                d = acc - cov_v[pl.ds(j * 128 + g * _LANES, _LANES)]
                return loss + d * d
            return group_body

        inflight = [fire(0), fire(1)]
        loss = jnp.zeros((_LANES,), jnp.float32)
        for j in range(n_blocks):
            for cp in inflight.pop(0):
                cp.wait()
            loss = lax.fori_loop(0, 128 // _LANES,
                                 make_group_body(lbufs[j % 2], rbufs[j % 2],
                                                 j), loss)
            if j + 2 < n_blocks:
                inflight.append(fire(j + 2))
        loss_v[...] = loss
        pltpu.sync_copy(loss_v, out_hbm.at[w])

    return scan_kernel, dot_kernel


def kernel(left, right, covariances, table):
    batch = left.shape[0]
    size, dim = table.shape
    nw = 32
    scan_kernel, dot_kernel = _make_kernels(batch, size)
    tail = size % _SROW
    tablet = table.T
    aux = jnp.pad(table[size - tail:].T,
                  ((0, 0), (0, _SROW - tail)))
    left = left.astype(jnp.int32)
    right = right.astype(jnp.int32)
    gl, gr = scan_kernel(left, right, tablet, aux)
    cov2 = covariances.reshape(nw, batch // nw)
    partials = dot_kernel(gl, gr, cov2)
    return jnp.sum(partials) / batch


# speedup vs baseline: 1.4314x; 1.4314x over previous
import functools

import jax
import jax.numpy as jnp
from jax import lax
from jax.experimental import pallas as pl
from jax.experimental.pallas import tpu as pltpu
from jax.experimental.pallas import tpu_sc as plsc

_DIM = 32
_LANES = 16
_CH = 1024
_WIN = 2048
_SROW = 128
_SEG = 2048
_SBITS = 15


def _make_kernels(batch, size):
    info = plsc.get_sparse_core_info()
    nc, ns = info.num_cores, info.num_subcores
    nw = nc * ns
    b_per_w = batch // nw
    tail = size % _SROW
    main = size - tail
    rng = 244 * _SROW
    extra = main - nw * rng
    n_win = rng // _WIN
    half = rng - n_win * _WIN
    stage_n = batch // 2
    dummy = batch
    smask = (1 << _SBITS) - 1

    mesh = plsc.VectorSubcoreMesh(core_axis_name="c", subcore_axis_name="s")

    @functools.partial(
        pl.kernel,
        mesh=mesh,
        out_type=(jax.ShapeDtypeStruct((batch + 1, _SROW), jnp.float32),
                  jax.ShapeDtypeStruct((batch + 1, _SROW), jnp.float32)),
        compiler_params=pltpu.CompilerParams(needs_layout_passes=False),
        scratch_types=[
            pltpu.VMEM((stage_n,), jnp.int32),
            pltpu.VMEM((batch,), jnp.int32),
            pltpu.VMEM((batch,), jnp.int32),
            pltpu.VMEM((2, _DIM, _CH), jnp.float32),
            pltpu.VMEM((_LANES, _SROW), jnp.float32),
            pltpu.VMEM((_LANES, _SROW), jnp.float32),
            pltpu.VMEM((_LANES,), jnp.int32),
            pltpu.VMEM((_LANES,), jnp.int32),
            pltpu.VMEM((_SEG,), jnp.int32),
            pltpu.SemaphoreType.DMA,
            pltpu.SemaphoreType.DMA,
            pltpu.SemaphoreType.DMA,
        ],
    )
    def scan_kernel(left_hbm, right_hbm, tablet_hbm, aux_hbm,
                    gl_hbm, gr_hbm,
                    stage_v, cl_v, cr_v, cb_v, rsA_v, rsB_v,
                    slA_v, slB_v, mini_v, sem_in, semA, semB):
        w = lax.axis_index("s") * nc + lax.axis_index("c")
        last = w == nw - 1
        lo = w * rng
        hi = jnp.where(last, jnp.int32(size), lo + rng)
        lanev = lax.iota(jnp.int32, _LANES)

        def chunk_src(j):
            off = pl.multiple_of(lo + j * _CH, _SROW)
            return tablet_hbm.at[:, pl.ds(off, _CH)]

        def compact(src_hbm, list_v):
            def stage_body(t, n):
                pltpu.sync_copy(src_hbm.at[pl.ds(t * stage_n, stage_n)],
                                stage_v)

                def body(v, n):
                    at = v * (4 * _LANES)
                    for k in range(4):
                        vec = stage_v[pl.ds(at + k * _LANES, _LANES)]
                        m = (vec >= lo) & (vec < hi)
                        slot = t * stage_n + at + k * _LANES + lanev
                        packed = ((vec - lo) << _SBITS) | slot
                        pos = n + plsc.cumsum(m.astype(jnp.int32)) - 1
                        plsc.store_scatter(list_v, [pos], packed, mask=m)
                        n = n + plsc.all_reduce_population_count(m)
                    return n

                return lax.fori_loop(0, stage_n // (4 * _LANES), body, n)

            n = lax.fori_loop(0, batch // stage_n, stage_body,
                              jnp.zeros((_LANES,), jnp.int32))
            return n[0]

        n_l = compact(left_hbm, cl_v)
        n_r = compact(right_hbm, cr_v)

        def rescan(list_v, n_list, g_hbm, buf, base_rel, width, carry):
            def seg_body(s, carry):
                seg = s * _SEG

                def match_body(v, q):
                    at = seg + v * (4 * _LANES)
                    for k in range(4):
                        vec = list_v[pl.ds(at + k * _LANES, _LANES)]
                        rel = vec >> _SBITS
                        valid = (at + k * _LANES + lanev) < n_list
                        m = ((rel >= base_rel) & (rel < base_rel + width)
                             & valid)
                        pos = q + plsc.cumsum(m.astype(jnp.int32)) - 1
                        plsc.store_scatter(mini_v, [pos], vec, mask=m)
                        q = q + plsc.all_reduce_population_count(m)
                    return q

                left_n = n_list - seg
                n_iters = jnp.clip(
                    (left_n + 4 * _LANES - 1) // (4 * _LANES),
                    0, _SEG // (4 * _LANES))
                q = lax.fori_loop(0, n_iters, match_body,
                                  jnp.zeros((_LANES,), jnp.int32))[0]

                def extract_body(gq, carry):
                    sA, sB = carry
                    packed = mini_v[pl.ds(gq * _LANES, _LANES)]
                    valid = (gq * _LANES + lanev) < q
                    cols = jnp.where(valid, (packed >> _SBITS) - base_rel,
                                     0)
                    slot = jnp.where(valid, packed & smask, dummy)
                    hv = cols >> 10
                    cv = cols & (_CH - 1)
                    even = ((sA + sB) % 2) == 0

                    def extract(rs_v, sl_v, sem):
                        for c in range(_DIM):
                            vals = plsc.load_gather(
                                buf,
                                [hv, jnp.full((_LANES,), c, jnp.int32), cv])
                            plsc.store_scatter(
                                rs_v,
                                [lanev, jnp.full((_LANES,), c, jnp.int32)],
                                vals)
                        sl_v[...] = slot
                        pltpu.async_copy(rs_v, g_hbm.at[sl_v], sem)

                    @pl.when(even)
                    def _():
                        @pl.when(sA >= 1)
                        def _():
                            pltpu.make_async_copy(
                                rsA_v, g_hbm.at[pl.ds(0, _LANES)],
                                semA).wait()
                        extract(rsA_v, slA_v, semA)

                    @pl.when(jnp.logical_not(even))
                    def _():
                        @pl.when(sB >= 1)
                        def _():
                            pltpu.make_async_copy(
                                rsB_v, g_hbm.at[pl.ds(0, _LANES)],
                                semB).wait()
                        extract(rsB_v, slB_v, semB)

                    sA = sA + jnp.where(even, 1, 0)
                    sB = sB + jnp.where(even, 0, 1)
                    return (sA, sB)

                n_groups = (q + _LANES - 1) // _LANES
                return lax.fori_loop(0, n_groups, extract_body, carry)

            n_segs = (n_list + _SEG - 1) // _SEG
            return lax.fori_loop(0, n_segs, seg_body, carry)

        def win_body(jw, carry):
            base_rel = jw * _WIN
            pltpu.async_copy(chunk_src(2 * jw), cb_v.at[0], sem_in)
            pltpu.async_copy(chunk_src(2 * jw + 1), cb_v.at[1], sem_in)
            pltpu.make_async_copy(chunk_src(0), cb_v.at[0], sem_in).wait()
            pltpu.make_async_copy(chunk_src(0), cb_v.at[1], sem_in).wait()
            carry = rescan(cl_v, n_l, gl_hbm, cb_v, base_rel, _WIN, carry)
            carry = rescan(cr_v, n_r, gr_hbm, cb_v, base_rel, _WIN, carry)
            return carry

        carry = lax.fori_loop(0, n_win, win_body,
                              (jnp.int32(0), jnp.int32(0)))

        off = pl.multiple_of(lo + n_win * _WIN, _SROW)
        pltpu.sync_copy(tablet_hbm.at[:, pl.ds(off, half)],
                        cb_v.at[0].at[:, pl.ds(0, half)])
        carry = rescan(cl_v, n_l, gl_hbm, cb_v,
                       jnp.int32(n_win * _WIN), jnp.int32(half), carry)
        carry = rescan(cr_v, n_r, gr_hbm, cb_v,
                       jnp.int32(n_win * _WIN), jnp.int32(half), carry)

        @pl.when(last)
        def _():
            pltpu.sync_copy(
                tablet_hbm.at[:, pl.ds(pl.multiple_of(nw * rng, _SROW),
                                       extra)],
                cb_v.at[0].at[:, pl.ds(0, extra)])

        ex_carry = carry
        for g_hbm, list_v, n_list in ((gl_hbm, cl_v, n_l),
                                      (gr_hbm, cr_v, n_r)):
            ex_carry = rescan(list_v, jnp.where(last, n_list, 0), g_hbm,
                              cb_v, jnp.int32(rng), jnp.int32(extra),
                              ex_carry)

        @pl.when(last)
        def _():
            pltpu.sync_copy(aux_hbm, cb_v.at[0].at[:, pl.ds(0, _SROW)])

        for g_hbm, list_v, n_list in ((gl_hbm, cl_v, n_l),
                                      (gr_hbm, cr_v, n_r)):
            ex_carry = rescan(list_v, jnp.where(last, n_list, 0), g_hbm,
                              cb_v, jnp.int32(rng + extra),
                              jnp.int32(tail), ex_carry)
        sA, sB = ex_carry

        @pl.when(sA >= 1)
        def _():
            pltpu.make_async_copy(rsA_v, gl_hbm.at[pl.ds(0, _LANES)],
                                  semA).wait()

        @pl.when(sB >= 1)
        def _():
            pltpu.make_async_copy(rsB_v, gl_hbm.at[pl.ds(0, _LANES)],
                                  semB).wait()

    n_blocks = b_per_w // 128

    @functools.partial(
        pl.kernel,
        mesh=mesh,
        out_type=jax.ShapeDtypeStruct((nw, _LANES), jnp.float32),
        compiler_params=pltpu.CompilerParams(needs_layout_passes=False),
        scratch_types=[
            pltpu.VMEM((b_per_w,), jnp.float32),
            pltpu.VMEM((128, _SROW), jnp.float32),
            pltpu.VMEM((128, _SROW), jnp.float32),
            pltpu.VMEM((128, _SROW), jnp.float32),
            pltpu.VMEM((128, _SROW), jnp.float32),
            pltpu.VMEM((_LANES,), jnp.float32),
            pltpu.SemaphoreType.DMA,
        ],
    )
    def dot_kernel(gl_hbm, gr_hbm, cov_hbm, out_hbm,
                   cov_v, la_v, lb_v, ra_v, rb_v, loss_v, sem):
        w = lax.axis_index("s") * nc + lax.axis_index("c")
        pltpu.sync_copy(cov_hbm.at[w], cov_v)
        lbufs = (la_v, lb_v)
        rbufs = (ra_v, rb_v)
        lanev = lax.iota(jnp.int32, _LANES)

        def fire(j):
            base = w * b_per_w + j * 128
            return (pltpu.async_copy(gl_hbm.at[pl.ds(base, 128)],
                                     lbufs[j % 2], sem),
                    pltpu.async_copy(gr_hbm.at[pl.ds(base, 128)],
                                     rbufs[j % 2], sem))

        def make_group_body(lbuf, rbuf, j):
            def group_body(g, loss):
                row = g * _LANES + lanev
                acc = jnp.zeros((_LANES,), jnp.float32)
                for c in range(_DIM):
                    col = jnp.full((_LANES,), c, jnp.int32)
                    lv = plsc.load_gather(lbuf, [row, col])
                    rv = plsc.load_gather(rbuf, [row, col])
                    acc = acc + lv * rv
                d = acc - cov_v[pl.ds(j * 128 + g * _LANES, _LANES)]
                return loss + d * d
            return group_body

        inflight = [fire(0), fire(1)]
        loss = jnp.zeros((_LANES,), jnp.float32)
        for j in range(n_blocks):
            for cp in inflight.pop(0):
                cp.wait()
            loss = lax.fori_loop(0, 128 // _LANES,
                                 make_group_body(lbufs[j % 2], rbufs[j % 2],
                                                 j), loss)
            if j + 2 < n_blocks:
                inflight.append(fire(j + 2))
        loss_v[...] = loss
        pltpu.sync_copy(loss_v, out_hbm.at[w])

    return scan_kernel, dot_kernel


def kernel(left, right, covariances, table):
    batch = left.shape[0]
    size, dim = table.shape
    nw = 32
    scan_kernel, dot_kernel = _make_kernels(batch, size)
    tail = size % _SROW
    tablet = table.T
    aux = jnp.pad(table[size - tail:].T,
                  ((0, 0), (0, _SROW - tail)))
    left = left.astype(jnp.int32)
    right = right.astype(jnp.int32)
    gl, gr = scan_kernel(left, right, tablet, aux)
    cov2 = covariances.reshape(nw, batch // nw)
    partials = dot_kernel(gl, gr, cov2)
    return jnp.sum(partials) / batch
